# baseline (speedup 1.0000x reference)
import jax
import jax.numpy as jnp
from jax.experimental import pallas as pl

_EPS = 1e-06
_F32 = jnp.float32


def _rsum24(a):
    return jnp.sum(a[0:8] + a[8:16] + a[16:24], axis=0, keepdims=True)


def _rep4(a):
    return jnp.concatenate(
        [jnp.broadcast_to(a[i : i + 1], (4,) + a.shape[1:]) for i in range(4)],
        axis=0,
    )


def _tile4(a):
    return jnp.concatenate([a, a, a, a], axis=0)


def _roll4(a):
    return jnp.concatenate([a[1:4], a[0:1]], axis=0)


def _inbox(ax, ay, bx, by, bx2, by2):
    xx = _rep4(bx) - _tile4(ax)
    yy = _rep4(by) - _tile4(ay)
    ex = _rep4(bx2 - bx)
    ey = _rep4(by2 - by)
    m1 = ((yy + ey) > 0.0) ^ (yy > 0.0)
    m2 = (xx - yy * ex / ey) > 0.0
    m = (m1 & m2).astype(jnp.int32)
    csum = m[0:4] + m[4:8] + m[8:12] + m[12:16]
    return (csum & 1) == 1


def _iou_kernel(b1_ref, b2_ref, out_ref):
    bt1 = jnp.transpose(b1_ref[...], (1, 0))
    bt2 = jnp.transpose(b2_ref[...], (1, 0))
    P = bt1.shape[1]

    x1, y1, w1, h1, a1 = (bt1[k : k + 1] for k in range(5))
    x2, y2, w2, h2, a2 = (bt2[k : k + 1] for k in range(5))

    half_pi = jnp.float32(1.5707963267948966)
    trig = jnp.sin(
        jnp.concatenate([a1, a2, a1 + half_pi, a2 + half_pi], axis=0)
    )
    sin1, sin2, cos1, cos2 = (trig[k : k + 1] for k in range(4))

    def corners(x, y, w, h, sin, cos):
        hw = 0.5 * w
        hh = 0.5 * h
        dx = jnp.concatenate([hw, -hw, -hw, hw], axis=0)
        dy = jnp.concatenate([hh, hh, -hh, -hh], axis=0)
        cx = x + dx * cos - dy * sin
        cy = y + dx * sin + dy * cos
        return cx, cy

    c1x, c1y = corners(x1, y1, w1, h1, sin1, cos1)
    c2x, c2y = corners(x2, y2, w2, h2, sin2, cos2)
    c1x2, c1y2 = _roll4(c1x), _roll4(c1y)
    c2x2, c2y2 = _roll4(c2x), _roll4(c2y)

    E1x, E1y = _rep4(c1x2 - c1x), _rep4(c1y2 - c1y)
    E2x, E2y = _tile4(c2x2 - c2x), _tile4(c2y2 - c2y)
    X1, Y1 = _rep4(c1x), _rep4(c1y)
    D31x = X1 - _tile4(c2x)
    D31y = Y1 - _tile4(c2y)

    num = E1x * E2y - E1y * E2x
    den_t = D31y * E2x - D31x * E2y
    t = den_t / num
    mask_t = (t >= 0.0) & (t <= 1.0)
    den_u = E1y * D31x - E1x * D31y
    u = -den_u / num
    mask_u = (u >= 0.0) & (u <= 1.0)
    mask_i = mask_t & mask_u
    t2 = den_t / (num + _EPS)
    mf_i = mask_i.astype(_F32)
    ix = (X1 + t2 * E1x) * mf_i
    iy = (Y1 + t2 * E1y) * mf_i

    c12 = _inbox(c1x, c1y, c2x, c2y, c2x2, c2y2)
    c21 = _inbox(c2x, c2y, c1x, c1y, c1x2, c1y2)

    vx = jnp.concatenate([c1x, c2x, ix], axis=0)
    vy = jnp.concatenate([c1y, c2y, iy], axis=0)
    mf = jnp.concatenate(
        [c12.astype(_F32), c21.astype(_F32), mf_i], axis=0
    )

    nv = _rsum24(mf)
    denom = jnp.maximum(nv, 1.0)
    mx = _rsum24(vx * mf) / denom
    my = _rsum24(vy * mf) / denom

    ang = jnp.arctan2(vy - my, vx - mx)
    ang = jnp.where(mf > 0.5, ang, 1e6)

    rows = jax.lax.broadcasted_iota(jnp.int32, (24, P), 0)
    bits = jax.lax.bitcast_convert_type(ang, jnp.int32)
    mono = bits ^ ((bits >> 31) & jnp.int32(0x7FFFFFFF))
    key = (mono & jnp.int32(~0x1F)) | rows

    big = jnp.int32(0x7FFFFFFF)
    sx = []
    sy = []
    kcur = key
    for p in range(8):
        m8 = jnp.minimum(jnp.minimum(kcur[0:8], kcur[8:16]), kcur[16:24])
        mk = jnp.min(m8, axis=0, keepdims=True)
        eq = kcur == mk
        sx.append(_rsum24(jnp.where(eq, vx, 0.0)))
        sy.append(_rsum24(jnp.where(eq, vy, 0.0)))
        if p < 7:
            kcur = jnp.where(eq, big, kcur)

    selx = [jnp.where(nv > float(p), sx[p], sx[0]) for p in range(8)]
    sely = [jnp.where(nv > float(p), sy[p], sy[0]) for p in range(8)]
    selx.append(sx[0])
    sely.append(sy[0])

    total = jnp.zeros_like(nv)
    for p in range(8):
        total = total + (selx[p] * sely[p + 1] - sely[p] * selx[p + 1])
    inter_area = jnp.abs(total) * 0.5

    area1 = w1 * h1
    area2 = w2 * h2
    union = area1 + area2 - inter_area
    out_ref[...] = inter_area / union


def kernel(box1, box2):
    B, N, _ = box1.shape
    T = B * N
    P = 4096

    b1 = jnp.pad(box1.reshape(T, 5).astype(_F32), ((0, 0), (0, 3)))
    b2 = jnp.pad(box2.reshape(T, 5).astype(_F32), ((0, 0), (0, 3)))

    out = pl.pallas_call(
        _iou_kernel,
        grid=((T + P - 1) // P,),
        in_specs=[
            pl.BlockSpec((P, 8), lambda i: (i, 0)),
            pl.BlockSpec((P, 8), lambda i: (i, 0)),
        ],
        out_specs=pl.BlockSpec((1, P), lambda i: (0, i)),
        out_shape=jax.ShapeDtypeStruct((1, T), _F32),
    )(b1, b2)

    return out[0].reshape(B, N)


# speedup vs baseline: 1.4069x; 1.4069x over previous
import jax
import jax.numpy as jnp
from jax.experimental import pallas as pl

_EPS = 1e-06
_F32 = jnp.float32


def _rsum24(a):
    return jnp.sum(a[0:8] + a[8:16] + a[16:24], axis=0, keepdims=True)


def _rep4(a):
    return jnp.concatenate(
        [jnp.broadcast_to(a[i : i + 1], (4,) + a.shape[1:]) for i in range(4)],
        axis=0,
    )


def _tile4(a):
    return jnp.concatenate([a, a, a, a], axis=0)


def _roll4(a):
    return jnp.concatenate([a[1:4], a[0:1]], axis=0)


def _inbox(ax, ay, bx, by, bx2, by2):
    xx = _rep4(bx) - _tile4(ax)
    yy = _rep4(by) - _tile4(ay)
    ex = _rep4(bx2 - bx)
    ey = _rep4(by2 - by)
    m1 = ((yy + ey) > 0.0) ^ (yy > 0.0)
    m2 = (xx - yy * ex / ey) > 0.0
    m = (m1 & m2).astype(jnp.int32)
    csum = m[0:4] + m[4:8] + m[8:12] + m[12:16]
    return (csum & 1) == 1


def _iou_kernel(b1_ref, b2_ref, out_ref):
    bt1 = jnp.transpose(b1_ref[...], (1, 0))
    bt2 = jnp.transpose(b2_ref[...], (1, 0))
    P = bt1.shape[1]

    x1, y1, w1, h1, a1 = (bt1[k : k + 1] for k in range(5))
    x2, y2, w2, h2, a2 = (bt2[k : k + 1] for k in range(5))

    half_pi = jnp.float32(1.5707963267948966)
    trig = jnp.sin(
        jnp.concatenate([a1, a2, a1 + half_pi, a2 + half_pi], axis=0)
    )
    sin1, sin2, cos1, cos2 = (trig[k : k + 1] for k in range(4))

    def corners(x, y, w, h, sin, cos):
        hw = 0.5 * w
        hh = 0.5 * h
        dx = jnp.concatenate([hw, -hw, -hw, hw], axis=0)
        dy = jnp.concatenate([hh, hh, -hh, -hh], axis=0)
        cx = x + dx * cos - dy * sin
        cy = y + dx * sin + dy * cos
        return cx, cy

    c1x, c1y = corners(x1, y1, w1, h1, sin1, cos1)
    c2x, c2y = corners(x2, y2, w2, h2, sin2, cos2)
    c1x2, c1y2 = _roll4(c1x), _roll4(c1y)
    c2x2, c2y2 = _roll4(c2x), _roll4(c2y)

    E1x, E1y = _rep4(c1x2 - c1x), _rep4(c1y2 - c1y)
    E2x, E2y = _tile4(c2x2 - c2x), _tile4(c2y2 - c2y)
    X1, Y1 = _rep4(c1x), _rep4(c1y)
    D31x = X1 - _tile4(c2x)
    D31y = Y1 - _tile4(c2y)

    num = E1x * E2y - E1y * E2x
    den_t = D31y * E2x - D31x * E2y
    t = den_t / num
    mask_t = (t >= 0.0) & (t <= 1.0)
    den_u = E1y * D31x - E1x * D31y
    u = -den_u / num
    mask_u = (u >= 0.0) & (u <= 1.0)
    mask_i = mask_t & mask_u
    t2 = den_t / (num + _EPS)
    mf_i = mask_i.astype(_F32)
    ix = (X1 + t2 * E1x) * mf_i
    iy = (Y1 + t2 * E1y) * mf_i

    c12 = _inbox(c1x, c1y, c2x, c2y, c2x2, c2y2)
    c21 = _inbox(c2x, c2y, c1x, c1y, c1x2, c1y2)

    vx = jnp.concatenate([c1x, c2x, ix], axis=0)
    vy = jnp.concatenate([c1y, c2y, iy], axis=0)
    mf = jnp.concatenate(
        [c12.astype(_F32), c21.astype(_F32), mf_i], axis=0
    )

    nv = _rsum24(mf)
    denom = jnp.maximum(nv, 1.0)
    mx = _rsum24(vx * mf) / denom
    my = _rsum24(vy * mf) / denom

    ang = jnp.arctan2(vy - my, vx - mx)
    ang = jnp.where(mf > 0.5, ang, 1e6)

    rows = jax.lax.broadcasted_iota(jnp.int32, (24, P), 0)
    bits = jax.lax.bitcast_convert_type(ang, jnp.int32)
    mono = bits ^ ((bits >> 31) & jnp.int32(0x7FFFFFFF))
    key = (mono & jnp.int32(~0x1F)) | rows

    big = jnp.int32(0x7FFFFFFF)
    sx = []
    sy = []
    kcur = key
    for p in range(8):
        m8 = jnp.minimum(jnp.minimum(kcur[0:8], kcur[8:16]), kcur[16:24])
        mk = jnp.min(m8, axis=0, keepdims=True)
        eq = kcur == mk
        sx.append(_rsum24(jnp.where(eq, vx, 0.0)))
        sy.append(_rsum24(jnp.where(eq, vy, 0.0)))
        if p < 7:
            kcur = jnp.where(eq, big, kcur)

    selx = [jnp.where(nv > float(p), sx[p], sx[0]) for p in range(8)]
    sely = [jnp.where(nv > float(p), sy[p], sy[0]) for p in range(8)]
    selx.append(sx[0])
    sely.append(sy[0])

    total = jnp.zeros_like(nv)
    for p in range(8):
        total = total + (selx[p] * sely[p + 1] - sely[p] * selx[p + 1])
    inter_area = jnp.abs(total) * 0.5

    area1 = w1 * h1
    area2 = w2 * h2
    union = area1 + area2 - inter_area
    out_ref[...] = inter_area / union


def kernel(box1, box2):
    B, N, _ = box1.shape
    T = B * N
    P = 4096

    b1 = box1.reshape(T, 5).astype(_F32)
    b2 = box2.reshape(T, 5).astype(_F32)

    out = pl.pallas_call(
        _iou_kernel,
        grid=((T + P - 1) // P,),
        in_specs=[
            pl.BlockSpec((P, 5), lambda i: (i, 0)),
            pl.BlockSpec((P, 5), lambda i: (i, 0)),
        ],
        out_specs=pl.BlockSpec((1, P), lambda i: (0, i)),
        out_shape=jax.ShapeDtypeStruct((1, T), _F32),
    )(b1, b2)

    return out[0].reshape(B, N)


# speedup vs baseline: 1.6905x; 1.2015x over previous
import jax
import jax.numpy as jnp
from jax.experimental import pallas as pl

_EPS = 1e-06
_F32 = jnp.float32


def _rsum24(a):
    return jnp.sum(a[0:8] + a[8:16] + a[16:24], axis=0, keepdims=True)


def _rep4(a):
    return jnp.concatenate(
        [jnp.broadcast_to(a[i : i + 1], (4,) + a.shape[1:]) for i in range(4)],
        axis=0,
    )


def _tile4(a):
    return jnp.concatenate([a, a, a, a], axis=0)


def _roll4(a):
    return jnp.concatenate([a[1:4], a[0:1]], axis=0)


def _inbox(ax, ay, bx, by, bx2, by2):
    xx = _rep4(bx) - _tile4(ax)
    yy = _rep4(by) - _tile4(ay)
    ex = _rep4(bx2 - bx)
    ey = _rep4(by2 - by)
    m1 = ((yy + ey) > 0.0) ^ (yy > 0.0)
    m2 = (xx - yy * ex / ey) > 0.0
    m = (m1 & m2).astype(jnp.int32)
    csum = m[0:4] + m[4:8] + m[8:12] + m[12:16]
    return (csum & 1) == 1


def _iou_kernel(
    x1_ref, y1_ref, w1_ref, h1_ref, a1_ref,
    x2_ref, y2_ref, w2_ref, h2_ref, a2_ref,
    out_ref,
):
    x1, y1, w1, h1, a1 = (
        r[...] for r in (x1_ref, y1_ref, w1_ref, h1_ref, a1_ref)
    )
    x2, y2, w2, h2, a2 = (
        r[...] for r in (x2_ref, y2_ref, w2_ref, h2_ref, a2_ref)
    )
    P = x1.shape[1]

    half_pi = jnp.float32(1.5707963267948966)
    trig = jnp.sin(
        jnp.concatenate([a1, a2, a1 + half_pi, a2 + half_pi], axis=0)
    )
    sin1, sin2, cos1, cos2 = (trig[k : k + 1] for k in range(4))

    def corners(x, y, w, h, sin, cos):
        hw = 0.5 * w
        hh = 0.5 * h
        dx = jnp.concatenate([hw, -hw, -hw, hw], axis=0)
        dy = jnp.concatenate([hh, hh, -hh, -hh], axis=0)
        cx = x + dx * cos - dy * sin
        cy = y + dx * sin + dy * cos
        return cx, cy

    c1x, c1y = corners(x1, y1, w1, h1, sin1, cos1)
    c2x, c2y = corners(x2, y2, w2, h2, sin2, cos2)
    c1x2, c1y2 = _roll4(c1x), _roll4(c1y)
    c2x2, c2y2 = _roll4(c2x), _roll4(c2y)

    E1x, E1y = _rep4(c1x2 - c1x), _rep4(c1y2 - c1y)
    E2x, E2y = _tile4(c2x2 - c2x), _tile4(c2y2 - c2y)
    X1, Y1 = _rep4(c1x), _rep4(c1y)
    D31x = X1 - _tile4(c2x)
    D31y = Y1 - _tile4(c2y)

    num = E1x * E2y - E1y * E2x
    den_t = D31y * E2x - D31x * E2y
    t = den_t / num
    mask_t = (t >= 0.0) & (t <= 1.0)
    den_u = E1y * D31x - E1x * D31y
    u = -den_u / num
    mask_u = (u >= 0.0) & (u <= 1.0)
    mask_i = mask_t & mask_u
    t2 = den_t / (num + _EPS)
    mf_i = mask_i.astype(_F32)
    ix = (X1 + t2 * E1x) * mf_i
    iy = (Y1 + t2 * E1y) * mf_i

    c12 = _inbox(c1x, c1y, c2x, c2y, c2x2, c2y2)
    c21 = _inbox(c2x, c2y, c1x, c1y, c1x2, c1y2)

    vx = jnp.concatenate([c1x, c2x, ix], axis=0)
    vy = jnp.concatenate([c1y, c2y, iy], axis=0)
    mf = jnp.concatenate(
        [c12.astype(_F32), c21.astype(_F32), mf_i], axis=0
    )

    nv = _rsum24(mf)
    denom = jnp.maximum(nv, 1.0)
    mx = _rsum24(vx * mf) / denom
    my = _rsum24(vy * mf) / denom

    ang = jnp.arctan2(vy - my, vx - mx)
    ang = jnp.where(mf > 0.5, ang, 1e6)

    rows = jax.lax.broadcasted_iota(jnp.int32, (24, P), 0)
    bits = jax.lax.bitcast_convert_type(ang, jnp.int32)
    mono = bits ^ ((bits >> 31) & jnp.int32(0x7FFFFFFF))
    key = (mono & jnp.int32(~0x1F)) | rows

    big = jnp.int32(0x7FFFFFFF)
    sx = []
    sy = []
    kcur = key
    for p in range(8):
        m8 = jnp.minimum(jnp.minimum(kcur[0:8], kcur[8:16]), kcur[16:24])
        mk = jnp.min(m8, axis=0, keepdims=True)
        eq = kcur == mk
        sx.append(_rsum24(jnp.where(eq, vx, 0.0)))
        sy.append(_rsum24(jnp.where(eq, vy, 0.0)))
        if p < 7:
            kcur = jnp.where(eq, big, kcur)

    selx = [jnp.where(nv > float(p), sx[p], sx[0]) for p in range(8)]
    sely = [jnp.where(nv > float(p), sy[p], sy[0]) for p in range(8)]
    selx.append(sx[0])
    sely.append(sy[0])

    total = jnp.zeros_like(nv)
    for p in range(8):
        total = total + (selx[p] * sely[p + 1] - sely[p] * selx[p + 1])
    inter_area = jnp.abs(total) * 0.5

    area1 = w1 * h1
    area2 = w2 * h2
    union = area1 + area2 - inter_area
    out_ref[...] = inter_area / union


def kernel(box1, box2):
    B, N, _ = box1.shape
    T = B * N
    P = 4096

    cols = [box1[:, :, k].reshape(1, T).astype(_F32) for k in range(5)]
    cols += [box2[:, :, k].reshape(1, T).astype(_F32) for k in range(5)]

    out = pl.pallas_call(
        _iou_kernel,
        grid=((T + P - 1) // P,),
        in_specs=[pl.BlockSpec((1, P), lambda i: (0, i)) for _ in range(10)],
        out_specs=pl.BlockSpec((1, P), lambda i: (0, i)),
        out_shape=jax.ShapeDtypeStruct((1, T), _F32),
    )(*cols)

    return out[0].reshape(B, N)


# speedup vs baseline: 2.1782x; 1.2885x over previous
import jax
import jax.numpy as jnp
from jax.experimental import pallas as pl

_EPS = 1e-06
_F32 = jnp.float32


def _rsum24(a):
    ones = jnp.ones((1, 24), dtype=a.dtype)
    return jax.lax.dot_general(
        ones, a, (((1,), (0,)), ((), ())),
        preferred_element_type=jnp.float32,
    )


def _rep4(a):
    return jnp.concatenate(
        [jnp.broadcast_to(a[i : i + 1], (4,) + a.shape[1:]) for i in range(4)],
        axis=0,
    )


def _tile4(a):
    return jnp.concatenate([a, a, a, a], axis=0)


def _roll4(a):
    return jnp.concatenate([a[1:4], a[0:1]], axis=0)


def _inbox(ax, ay, bx, by, bx2, by2):
    xx = _rep4(bx) - _tile4(ax)
    yy = _rep4(by) - _tile4(ay)
    ex = _rep4(bx2 - bx)
    ey = _rep4(by2 - by)
    m1 = ((yy + ey) > 0.0) ^ (yy > 0.0)
    m2 = (xx - yy * ex / ey) > 0.0
    m = (m1 & m2).astype(jnp.int32)
    csum = m[0:4] + m[4:8] + m[8:12] + m[12:16]
    return (csum & 1) == 1


def _iou_kernel(b1_ref, b2_ref, out_ref):
    b1 = b1_ref[...]
    b2 = b2_ref[...]
    P = b1.shape[1]

    x1, y1, w1, h1, a1 = (b1[k : k + 1] for k in range(5))
    x2, y2, w2, h2, a2 = (b2[k : k + 1] for k in range(5))

    half_pi = jnp.float32(1.5707963267948966)
    trig = jnp.sin(
        jnp.concatenate([a1, a2, a1 + half_pi, a2 + half_pi], axis=0)
    )
    sin1, sin2, cos1, cos2 = (trig[k : k + 1] for k in range(4))

    def corners(x, y, w, h, sin, cos):
        hw = 0.5 * w
        hh = 0.5 * h
        dx = jnp.concatenate([hw, -hw, -hw, hw], axis=0)
        dy = jnp.concatenate([hh, hh, -hh, -hh], axis=0)
        cx = x + dx * cos - dy * sin
        cy = y + dx * sin + dy * cos
        return cx, cy

    c1x, c1y = corners(x1, y1, w1, h1, sin1, cos1)
    c2x, c2y = corners(x2, y2, w2, h2, sin2, cos2)
    c1x2, c1y2 = _roll4(c1x), _roll4(c1y)
    c2x2, c2y2 = _roll4(c2x), _roll4(c2y)

    E1x, E1y = _rep4(c1x2 - c1x), _rep4(c1y2 - c1y)
    E2x, E2y = _tile4(c2x2 - c2x), _tile4(c2y2 - c2y)
    X1, Y1 = _rep4(c1x), _rep4(c1y)
    D31x = X1 - _tile4(c2x)
    D31y = Y1 - _tile4(c2y)

    num = E1x * E2y - E1y * E2x
    den_t = D31y * E2x - D31x * E2y
    t = den_t / num
    mask_t = (t >= 0.0) & (t <= 1.0)
    den_u = E1y * D31x - E1x * D31y
    u = -den_u / num
    mask_u = (u >= 0.0) & (u <= 1.0)
    mask_i = mask_t & mask_u
    t2 = den_t / (num + _EPS)
    mf_i = mask_i.astype(_F32)
    ix = (X1 + t2 * E1x) * mf_i
    iy = (Y1 + t2 * E1y) * mf_i

    c12 = _inbox(c1x, c1y, c2x, c2y, c2x2, c2y2)
    c21 = _inbox(c2x, c2y, c1x, c1y, c1x2, c1y2)

    vx = jnp.concatenate([c1x, c2x, ix], axis=0)
    vy = jnp.concatenate([c1y, c2y, iy], axis=0)
    mf = jnp.concatenate(
        [c12.astype(_F32), c21.astype(_F32), mf_i], axis=0
    )

    nv = _rsum24(mf)
    denom = jnp.maximum(nv, 1.0)
    mx = _rsum24(vx * mf) / denom
    my = _rsum24(vy * mf) / denom

    ang = jnp.arctan2(vy - my, vx - mx)
    ang = jnp.where(mf > 0.5, ang, 1e6)

    rows = jax.lax.broadcasted_iota(jnp.int32, (24, P), 0)
    bits = jax.lax.bitcast_convert_type(ang, jnp.int32)
    mono = bits ^ ((bits >> 31) & jnp.int32(0x7FFFFFFF))
    key = (mono & jnp.int32(~0x1F)) | rows

    big = jnp.int32(0x7FFFFFFF)
    sx = []
    sy = []
    kcur = key
    for p in range(8):
        m8 = jnp.minimum(jnp.minimum(kcur[0:8], kcur[8:16]), kcur[16:24])
        mk = jnp.min(m8, axis=0, keepdims=True)
        eq = kcur == mk
        sx.append(_rsum24(jnp.where(eq, vx, 0.0)))
        sy.append(_rsum24(jnp.where(eq, vy, 0.0)))
        if p < 7:
            kcur = jnp.where(eq, big, kcur)

    selx = [jnp.where(nv > float(p), sx[p], sx[0]) for p in range(8)]
    sely = [jnp.where(nv > float(p), sy[p], sy[0]) for p in range(8)]
    selx.append(sx[0])
    sely.append(sy[0])

    total = jnp.zeros_like(nv)
    for p in range(8):
        total = total + (selx[p] * sely[p + 1] - sely[p] * selx[p + 1])
    inter_area = jnp.abs(total) * 0.5

    area1 = w1 * h1
    area2 = w2 * h2
    union = area1 + area2 - inter_area
    out_ref[...] = inter_area / union


def kernel(box1, box2):
    B, N, _ = box1.shape
    T = B * N
    P = 2048

    Tpad = ((T + P - 1) // P) * P

    def prep(b):
        bt = b.reshape(T, 5).T.astype(_F32)
        return jnp.pad(bt, ((0, 3), (0, Tpad - T)))

    out = pl.pallas_call(
        _iou_kernel,
        grid=(Tpad // P,),
        in_specs=[
            pl.BlockSpec((8, P), lambda i: (0, i)),
            pl.BlockSpec((8, P), lambda i: (0, i)),
        ],
        out_specs=pl.BlockSpec((1, P), lambda i: (0, i)),
        out_shape=jax.ShapeDtypeStruct((1, Tpad), _F32),
    )(prep(box1), prep(box2))

    return out[0, :T].reshape(B, N)
